# baseline (speedup 1.0000x reference)
import jax
import jax.numpy as jnp
from jax import lax
from jax.experimental import pallas as pl
from jax.experimental.pallas import tpu as pltpu
from jax.experimental.pallas import tpu_sc as plsc

_L = 16
_BW = 128
_D = 64
_NBUF = 3


def _make_lookup(bsz, seq):
    info = plsc.get_sparse_core_info()
    NC, NS = info.num_cores, info.num_subcores
    NW = NC * NS
    assert bsz % (NW * _BW) == 0 and bsz // NW == _BW
    mesh = plsc.VectorSubcoreMesh(core_axis_name="c", subcore_axis_name="s")

    def body(xt_hbm, table_hbm, out_hbm, idx_v, pair_v, rows_v, tile_v,
             sems, wsems):
        wid = lax.axis_index("s") * NC + lax.axis_index("c")
        b0 = wid * _BW
        pltpu.sync_copy(xt_hbm.at[:, pl.ds(b0, _BW)], idx_v)

        def prep(s, buf):
            for gi in range(_BW // _L):
                v = idx_v[s, pl.ds(gi * _L, _L)]
                pair_v[buf, pl.ds(gi * _L, _L)] = jax.lax.shift_right_logical(v, 1)

        def gather(buf):
            pltpu.async_copy(table_hbm.at[pair_v.at[buf]],
                             rows_v.at[buf], sems.at[buf])

        for b in range(_NBUF - 1):
            prep(b, b)
            gather(b)

        rowc = [lax.iota(jnp.int32, _L) + gi * _L for gi in range(_BW // _L)]

        def step(s, carry):
            buf = lax.rem(s, _NBUF)
            pltpu.make_async_copy(table_hbm.at[pair_v.at[buf]],
                                  rows_v.at[buf], sems.at[buf]).wait()

            @pl.when(s + _NBUF - 1 < seq)
            def _():
                nbuf = lax.rem(s + _NBUF - 1, _NBUF)
                prep(s + _NBUF - 1, nbuf)
                gather(nbuf)

            cols0 = tuple(
                (idx_v[s, pl.ds(gi * _L, _L)] & 1) * _D
                for gi in range(_BW // _L)
            )

            def fstep(f, cols):
                vals = [
                    plsc.load_gather(rows_v.at[buf], [rowc[gi], cols[gi]])
                    for gi in range(_BW // _L)
                ]
                for gi in range(_BW // _L):
                    tile_v[buf, f, pl.ds(gi * _L, _L)] = vals[gi]
                return tuple(c + 1 for c in cols)

            lax.fori_loop(0, _D, fstep, cols0, unroll=4)

            pltpu.async_copy(tile_v.at[buf],
                             out_hbm.at[s, :, pl.ds(b0, _BW)],
                             wsems.at[buf])
            return carry

        def step_outer(s, carry):
            @pl.when(s >= _NBUF)
            def _():
                buf = lax.rem(s, _NBUF)
                pltpu.make_async_copy(
                    tile_v.at[buf],
                    out_hbm.at[s - _NBUF, :, pl.ds(b0, _BW)],
                    wsems.at[buf]).wait()
            return step(s, carry)

        lax.fori_loop(0, seq, step_outer, 0)

        for b in range(_NBUF):
            s_last = seq - _NBUF + b
            pltpu.make_async_copy(tile_v.at[b],
                                  out_hbm.at[s_last, :, pl.ds(b0, _BW)],
                                  wsems.at[b]).wait()

    return pl.kernel(
        body,
        out_type=jax.ShapeDtypeStruct((seq, _D, bsz), jnp.float32),
        mesh=mesh,
        scratch_types=[
            pltpu.VMEM((seq, _BW), jnp.int32),
            pltpu.VMEM((_NBUF, _BW), jnp.int32),
            pltpu.VMEM((_NBUF, _BW, 2 * _D), jnp.float32),
            pltpu.VMEM((_NBUF, _D, _BW), jnp.float32),
            pltpu.SemaphoreType.DMA((_NBUF,)),
            pltpu.SemaphoreType.DMA((_NBUF,)),
        ],
        compiler_params=pltpu.CompilerParams(use_tc_tiling_on_sc=True,
                                             needs_layout_passes=False),
    )


def kernel(x, table):
    bsz, seq = x.shape
    x_t = x.T.astype(jnp.int32)
    table_p = table.reshape(table.shape[0] // 2, 2 * _D)
    out_t = _make_lookup(bsz, seq)(x_t, table_p)
    return jnp.transpose(out_t, (2, 0, 1))


# speedup vs baseline: 1.2983x; 1.2983x over previous
import jax
import jax.numpy as jnp
from jax import lax
from jax.experimental import pallas as pl
from jax.experimental.pallas import tpu as pltpu
from jax.experimental.pallas import tpu_sc as plsc

_D = 64
_CHUNK = 512
_NBUF = 2


def _make_lookup(B):
    info = plsc.get_sparse_core_info()
    NC, NS = info.num_cores, info.num_subcores
    NW = NC * NS
    assert B % (NW * _CHUNK) == 0
    b_per_w = B // NW
    n_chunks = b_per_w // _CHUNK
    mesh = plsc.VectorSubcoreMesh(core_axis_name="c", subcore_axis_name="s")

    def body(x_hbm, table_hbm, out_hbm, idx_v, rows_v, sems):
        wid = lax.axis_index("s") * NC + lax.axis_index("c")
        pltpu.sync_copy(x_hbm.at[pl.ds(wid * b_per_w, b_per_w)], idx_v)

        def gather(j, b):
            pltpu.async_copy(table_hbm.at[idx_v.at[pl.ds(j * _CHUNK, _CHUNK)]],
                             rows_v.at[b], sems.at[b])

        for b in range(_NBUF):
            gather(b, b)

        def chunk(j, carry):
            b = lax.rem(j, _NBUF)
            pltpu.make_async_copy(table_hbm.at[idx_v.at[pl.ds(j * _CHUNK, _CHUNK)]],
                                  rows_v.at[b], sems.at[b]).wait()
            row0 = wid * b_per_w + j * _CHUNK
            pltpu.sync_copy(rows_v.at[b], out_hbm.at[pl.ds(row0, _CHUNK)])

            @pl.when(j + _NBUF < n_chunks)
            def _():
                gather(j + _NBUF, b)

            return carry

        lax.fori_loop(0, n_chunks, chunk, 0)

    return pl.kernel(
        body,
        out_type=jax.ShapeDtypeStruct((B, _D), jnp.float32),
        mesh=mesh,
        scratch_types=[
            pltpu.VMEM((b_per_w,), jnp.int32),
            pltpu.VMEM((_NBUF, _CHUNK, _D), jnp.float32),
            pltpu.SemaphoreType.DMA((_NBUF,)),
        ],
        compiler_params=pltpu.CompilerParams(use_tc_tiling_on_sc=False),
    )


def kernel(x, table):
    bsz, seq = x.shape
    B = bsz * seq
    x_flat = x.reshape(B).astype(jnp.int32)
    out = _make_lookup(B)(x_flat, table)
    return out.reshape(bsz, seq, _D)
